# Initial kernel scaffold; baseline (speedup 1.0000x reference)
import jax
import jax.numpy as jnp
from jax.experimental import pallas as pl


def kernel(X, at_time, at_src, at_dst, at_val, edge_time, edge_src, edge_trg, M, Minv, W1, W2, U):
    raise NotImplementedError("write your pallas kernel here")


# speedup vs baseline: 2.8243x; 2.8243x over previous
import functools

import jax
import jax.numpy as jnp
from jax import lax
from jax.experimental import pallas as pl
from jax.experimental.pallas import tpu as pltpu
from jax.experimental.pallas import tpu_sc as plsc

_T = 8
_N = 10000
_E = 320000
_F = 128
_R = _T * _N

_NC = 2
_NS = 16
_NW = _NC * _NS

_WIN = 12288
_PASS = _WIN * _NC
_NP = 4
_RPAD = _PASS * _NP
_ACC = 12416
_DUMP = _WIN
_CAP = 4096
_NCHK = _CAP // 128
_STRIP = _E // _NS
_SCH = 2000

_BN = 1000


def _round_bf(x):
    return x.astype(jnp.bfloat16).astype(jnp.float32)


def _mix_body(m_ref, x_ref, o_ref):
    xs = [_round_bf(x_ref[k]) for k in range(_T)]
    for t in range(_T):
        acc = xs[0] * m_ref[t, 0]
        for k in range(1, _T):
            acc = acc + xs[k] * m_ref[t, k]
        o_ref[t] = acc


def _mix(m, x):
    return pl.pallas_call(
        _mix_body,
        grid=(_N // _BN,),
        in_specs=[
            pl.BlockSpec(memory_space=pltpu.SMEM),
            pl.BlockSpec((_T, _BN, _F), lambda i: (0, i, 0)),
        ],
        out_specs=pl.BlockSpec((_T, _BN, _F), lambda i: (0, i, 0)),
        out_shape=jax.ShapeDtypeStruct((_T, _N, _F), jnp.float32),
    )(m, x)


def _tmix_terms(coef_ref, arrs, k):
    acc = arrs[0] * coef_ref[k, 0]
    for t in range(1, _T):
        acc = acc + arrs[t] * coef_ref[k, t]
    return acc


def _layer1_body(minv_ref, m_ref, s1_ref, w1_ref, o_ref):
    a = [_round_bf(jnp.dot(s1_ref[t].astype(jnp.bfloat16),
                           w1_ref[t].astype(jnp.bfloat16),
                           preferred_element_type=jnp.float32))
         for t in range(_T)]
    y = [_round_bf(jnp.maximum(_tmix_terms(minv_ref, a, k), 0.0))
         for k in range(_T)]
    for t in range(_T):
        o_ref[t] = _tmix_terms(m_ref, y, t)


def _layer1(minv, m, s1, w1):
    return pl.pallas_call(
        _layer1_body,
        grid=(_N // _BN,),
        in_specs=[
            pl.BlockSpec(memory_space=pltpu.SMEM),
            pl.BlockSpec(memory_space=pltpu.SMEM),
            pl.BlockSpec((_T, _BN, _F), lambda i: (0, i, 0)),
            pl.BlockSpec((_T, _F, _F), lambda i: (0, 0, 0)),
        ],
        out_specs=pl.BlockSpec((_T, _BN, _F), lambda i: (0, i, 0)),
        out_shape=jax.ShapeDtypeStruct((_T, _N, _F), jnp.float32),
    )(minv, m, s1, w1)


def _layer2_body(minv_ref, s2_ref, w2_ref, u0_ref, u1_ref, p_ref, q_ref):
    a = [_round_bf(jnp.dot(s2_ref[t].astype(jnp.bfloat16),
                           w2_ref[t].astype(jnp.bfloat16),
                           preferred_element_type=jnp.float32))
         for t in range(_T)]
    for k in range(_T):
        z = _tmix_terms(minv_ref, a, k).astype(jnp.bfloat16)
        p_ref[k] = jnp.dot(z, u0_ref[...], preferred_element_type=jnp.float32)
        q_ref[k] = jnp.dot(z, u1_ref[...], preferred_element_type=jnp.float32)


def _layer2(minv, s2, w2, u0, u1):
    return pl.pallas_call(
        _layer2_body,
        grid=(_N // _BN,),
        in_specs=[
            pl.BlockSpec(memory_space=pltpu.SMEM),
            pl.BlockSpec((_T, _BN, _F), lambda i: (0, i, 0)),
            pl.BlockSpec((_T, _F, _F), lambda i: (0, 0, 0)),
            pl.BlockSpec((_F, _F), lambda i: (0, 0)),
            pl.BlockSpec((_F, _F), lambda i: (0, 0)),
        ],
        out_specs=[
            pl.BlockSpec((_T, _BN, _F), lambda i: (0, i, 0)),
            pl.BlockSpec((_T, _BN, _F), lambda i: (0, i, 0)),
        ],
        out_shape=[
            jax.ShapeDtypeStruct((_T, _N, _F), jnp.float32),
            jax.ShapeDtypeStruct((_T, _N, _F), jnp.float32),
        ],
    )(minv, s2, w2, u0, u1)


def _sc_mesh():
    return plsc.VectorSubcoreMesh(core_axis_name="c", subcore_axis_name="s",
                                  num_cores=_NC, num_subcores=_NS)


def _bucketize_body(at_t, at_s, at_d, at_v, src_o, val_o, dst_o, cnt_o,
                    tbuf, sbuf, dbuf, vbuf,
                    srcb0, srcb1, srcb2, srcb3, dstb0, dstb1, dstb2, dstb3,
                    valb0, valb1, valb2, valb3, cvec):
    c = lax.axis_index("c")
    s = lax.axis_index("s")
    widx = c * _NS + s
    srcbs = (srcb0, srcb1, srcb2, srcb3)
    dstbs = (dstb0, dstb1, dstb2, dstb3)
    valbs = (valb0, valb1, valb2, valb3)

    zero_i = jnp.zeros((16,), jnp.int32)
    zero_f = jnp.zeros((16,), jnp.float32)
    dump = jnp.full((16,), _DUMP, jnp.int32)

    def prefill(i, _):
        for p in range(_NP):
            srcbs[p][pl.ds(i * 16, 16)] = zero_i
            dstbs[p][pl.ds(i * 16, 16)] = dump
            valbs[p][pl.ds(i * 16, 16)] = zero_f
        return 0

    lax.fori_loop(0, (_CAP + 16) // 16, prefill, 0)

    def scan_chunk(ic, cnts):
        base_e = s * _STRIP + ic * _SCH
        pltpu.sync_copy(at_t.at[pl.ds(base_e, _SCH)], tbuf)
        pltpu.sync_copy(at_s.at[pl.ds(base_e, _SCH)], sbuf)
        pltpu.sync_copy(at_d.at[pl.ds(base_e, _SCH)], dbuf)
        pltpu.sync_copy(at_v.at[pl.ds(base_e, _SCH)], vbuf)

        lanes = lax.iota(jnp.int32, 16)

        def vec(i, cnts):
            t = tbuf[pl.ds(i * 16, 16)]
            sg = t * _N + sbuf[pl.ds(i * 16, 16)]
            dg = t * _N + dbuf[pl.ds(i * 16, 16)]
            v = vbuf[pl.ds(i * 16, 16)]
            out = []
            for p in range(_NP):
                off = dg - (p * _PASS + c * _WIN)
                m = (off >= 0) & (off < _WIN)
                cnt = cnts[p]
                mi = m.astype(jnp.int32)
                inc = plsc.cumsum(mi)
                pos = jnp.where(m, cnt + inc - 1, _CAP + lanes)
                plsc.store_scatter(srcbs[p], [pos], sg)
                plsc.store_scatter(dstbs[p], [pos], off)
                plsc.store_scatter(valbs[p], [pos], v)
                out.append(cnt + jnp.sum(mi))
            return tuple(out)

        return lax.fori_loop(0, _SCH // 16, vec, cnts)

    z = jnp.zeros((), jnp.int32)
    cnts = lax.fori_loop(0, _STRIP // _SCH, scan_chunk, (z,) * _NP)

    lanes = lax.iota(jnp.int32, 16)
    cv = jnp.zeros((16,), jnp.int32)
    for p in range(_NP):
        b = widx * _NP + p
        pltpu.sync_copy(srcbs[p].at[pl.ds(0, _CAP)], src_o.at[b])
        pltpu.sync_copy(valbs[p].at[pl.ds(0, _CAP)], val_o.at[b])
        for ch in range(_NCHK):
            pltpu.sync_copy(dstbs[p].at[pl.ds(ch * 128, 128)], dst_o.at[b, ch])
        nch = (cnts[p] + 127) // 128
        cv = jnp.where(lanes == p, jnp.full((16,), nch, jnp.int32), cv)
    cvec[...] = cv
    pltpu.sync_copy(cvec, cnt_o.at[widx])


def _bucketize(at_t, at_s, at_d, at_v):
    f = pl.kernel(
        _bucketize_body,
        out_type=[
            jax.ShapeDtypeStruct((_NW * _NP, _CAP), jnp.int32),
            jax.ShapeDtypeStruct((_NW * _NP, _CAP), jnp.float32),
            jax.ShapeDtypeStruct((_NW * _NP, _NCHK, 128), jnp.int32),
            jax.ShapeDtypeStruct((_NW, 16), jnp.int32),
        ],
        mesh=_sc_mesh(),
        compiler_params=pltpu.CompilerParams(needs_layout_passes=False),
        scratch_types=[
            pltpu.VMEM((_SCH,), jnp.int32),
            pltpu.VMEM((_SCH,), jnp.int32),
            pltpu.VMEM((_SCH,), jnp.int32),
            pltpu.VMEM((_SCH,), jnp.float32),
        ] + [pltpu.VMEM((_CAP + 16,), jnp.int32)] * (2 * _NP)
          + [pltpu.VMEM((_CAP + 16,), jnp.float32)] * _NP
          + [pltpu.VMEM((16,), jnp.int32)],
    )
    return f(at_t, at_s, at_d, at_v)


def _spmm_body(xt, zeros_h, src_t, val_t, dst_t, cnt_t, out,
               srcv, valv, dstv, cntv, rows, acc, sem):
    c = lax.axis_index("c")
    s = lax.axis_index("s")
    widx = c * _NS + s

    pltpu.sync_copy(cnt_t.at[widx], cntv)
    cv = cntv[...]
    lanes = lax.iota(jnp.int32, 16)

    for p in range(_NP):
        b = widx * _NP + p
        pltpu.sync_copy(src_t.at[b], srcv)
        pltpu.sync_copy(val_t.at[b], valv)
        pltpu.sync_copy(dst_t.at[b], dstv)
        pltpu.sync_copy(zeros_h, acc.at[pl.ds(s * (_ACC // _NS), _ACC // _NS)])
        plsc.subcore_barrier()

        nch = jnp.sum(jnp.where(lanes == p, cv, 0))

        def chunk(j, _):
            pltpu.async_copy(xt.at[srcv.at[pl.ds(j * 128, 128)]], rows,
                             sem).wait()

            def scale(r, _):
                vb = plsc.load_gather(
                    valv, [jnp.full((16,), j * 128 + r, jnp.int32)])
                for f in range(8):
                    rows[r, pl.ds(f * 16, 16)] = rows[r, pl.ds(f * 16, 16)] * vb
                return 0

            lax.fori_loop(0, 128, scale, 0)
            pltpu.sync_copy(rows, acc.at[dstv.at[j]], add=True)
            return 0

        lax.fori_loop(0, nch, chunk, 0)
        plsc.subcore_barrier()

        tw = _WIN // _NS
        gbase = p * _PASS + c * _WIN + s * tw
        for i in range(tw // 128):
            pltpu.sync_copy(acc.at[pl.ds(s * tw + i * 128, 128)],
                            out.at[pl.ds(gbase + i * 128, 128)])
        plsc.subcore_barrier()


def _spmm(xt, zeros_h, src_t, val_t, dst_t, cnt_t):
    f = pl.kernel(
        _spmm_body,
        out_type=jax.ShapeDtypeStruct((_RPAD, _F), jnp.float32),
        mesh=_sc_mesh(),
        compiler_params=pltpu.CompilerParams(needs_layout_passes=False),
        scratch_types=[
            pltpu.VMEM((_CAP,), jnp.int32),
            pltpu.VMEM((_CAP,), jnp.float32),
            pltpu.VMEM((_NCHK, 128), jnp.int32),
            pltpu.VMEM((16,), jnp.int32),
            pltpu.VMEM((128, _F), jnp.float32),
            pltpu.VMEM_SHARED((_ACC, _F), jnp.float32),
            pltpu.SemaphoreType.DMA,
        ],
    )
    return f(xt, zeros_h, src_t, val_t, dst_t, cnt_t)


_EW = _E // _NW
_ECH = 80


def _edge_body(p_t, q_t, et, es, eg, out, tb, sb, gb, esn, etn, rows, sem):
    c = lax.axis_index("c")
    s = lax.axis_index("s")
    widx = c * _NS + s
    base = widx * _EW

    pltpu.sync_copy(et.at[pl.ds(base, _EW)], tb)
    pltpu.sync_copy(es.at[pl.ds(base, _EW)], sb)
    pltpu.sync_copy(eg.at[pl.ds(base, _EW)], gb)

    def cvt(i, _):
        t = tb[pl.ds(i * 16, 16)]
        esn[pl.ds(i * 16, 16)] = t * _N + sb[pl.ds(i * 16, 16)]
        etn[pl.ds(i * 16, 16)] = t * _N + gb[pl.ds(i * 16, 16)]
        return 0

    lax.fori_loop(0, _EW // 16, cvt, 0)

    def chunk(j, _):
        pltpu.async_copy(p_t.at[esn.at[pl.ds(j * _ECH, _ECH)]], rows,
                         sem).wait()
        pltpu.async_copy(q_t.at[etn.at[pl.ds(j * _ECH, _ECH)]], rows,
                         sem, add=True).wait()
        pltpu.sync_copy(rows, out.at[pl.ds(base + j * _ECH, _ECH)])
        return 0

    lax.fori_loop(0, _EW // _ECH, chunk, 0)


def _edge_out(p_t, q_t, et, es, eg):
    f = pl.kernel(
        _edge_body,
        out_type=jax.ShapeDtypeStruct((_E, _F), jnp.float32),
        mesh=_sc_mesh(),
        compiler_params=pltpu.CompilerParams(needs_layout_passes=False),
        scratch_types=[
            pltpu.VMEM((_EW,), jnp.int32),
            pltpu.VMEM((_EW,), jnp.int32),
            pltpu.VMEM((_EW,), jnp.int32),
            pltpu.VMEM((_EW,), jnp.int32),
            pltpu.VMEM((_EW,), jnp.int32),
            pltpu.VMEM((_ECH, _F), jnp.float32),
            pltpu.SemaphoreType.DMA,
        ],
    )
    return f(p_t, q_t, et, es, eg)


def kernel(X, at_time, at_src, at_dst, at_val, edge_time, edge_src, edge_trg,
           M, Minv, W1, W2, U):
    mr = lax.optimization_barrier(M.astype(jnp.bfloat16)).astype(jnp.float32)
    minvr = lax.optimization_barrier(
        Minv.astype(jnp.bfloat16)).astype(jnp.float32)
    xt = _mix(mr, X)
    zeros_h = jnp.zeros((_ACC // _NS, _F), jnp.float32)
    src_t, val_t, dst_t, cnt_t = _bucketize(at_time, at_src, at_dst, at_val)
    s1 = _spmm(xt.reshape(_R, _F), zeros_h, src_t, val_t, dst_t, cnt_t)
    yt = _layer1(minvr, mr, s1[:_R].reshape(_T, _N, _F), W1)
    s2 = _spmm(yt.reshape(_R, _F), zeros_h, src_t, val_t, dst_t, cnt_t)
    p, q = _layer2(minvr, s2[:_R].reshape(_T, _N, _F), W2,
                   U[:_F].astype(jnp.bfloat16), U[_F:].astype(jnp.bfloat16))
    return _edge_out(p.reshape(_R, _F), q.reshape(_R, _F),
                     edge_time, edge_src, edge_trg)


# speedup vs baseline: 3.1837x; 1.1273x over previous
import functools

import jax
import jax.numpy as jnp
from jax import lax
from jax.experimental import pallas as pl
from jax.experimental.pallas import tpu as pltpu
from jax.experimental.pallas import tpu_sc as plsc

_T = 8
_N = 10000
_E = 320000
_F = 128
_R = _T * _N

_NC = 2
_NS = 16
_NW = _NC * _NS

_WIN = 10240
_PASS = _WIN * _NC
_NP = 4
_RPAD = _PASS * _NP
_ACC = 10368
_DUMP = _WIN
_CAP = 3584
_NCHK = _CAP // 128
_STRIP = _E // _NS
_SCH = 2000

_BN = 1000


def _round_bf(x):
    return x.astype(jnp.bfloat16).astype(jnp.float32)


def _mix_body(m_ref, x_ref, o_ref):
    xs = [_round_bf(x_ref[k]) for k in range(_T)]
    for t in range(_T):
        acc = xs[0] * m_ref[t, 0]
        for k in range(1, _T):
            acc = acc + xs[k] * m_ref[t, k]
        o_ref[t] = acc


def _mix(m, x):
    return pl.pallas_call(
        _mix_body,
        grid=(_N // _BN,),
        in_specs=[
            pl.BlockSpec(memory_space=pltpu.SMEM),
            pl.BlockSpec((_T, _BN, _F), lambda i: (0, i, 0)),
        ],
        out_specs=pl.BlockSpec((_T, _BN, _F), lambda i: (0, i, 0)),
        out_shape=jax.ShapeDtypeStruct((_T, _N, _F), jnp.float32),
    )(m, x)


def _tmix_terms(coef_ref, arrs, k):
    acc = arrs[0] * coef_ref[k, 0]
    for t in range(1, _T):
        acc = acc + arrs[t] * coef_ref[k, t]
    return acc


def _layer1_body(minv_ref, m_ref, s1_ref, w1_ref, o_ref):
    a = [_round_bf(jnp.dot(s1_ref[t].astype(jnp.bfloat16),
                           w1_ref[t].astype(jnp.bfloat16),
                           preferred_element_type=jnp.float32))
         for t in range(_T)]
    y = [_round_bf(jnp.maximum(_tmix_terms(minv_ref, a, k), 0.0))
         for k in range(_T)]
    for t in range(_T):
        o_ref[t] = _tmix_terms(m_ref, y, t)


def _layer1(minv, m, s1, w1):
    return pl.pallas_call(
        _layer1_body,
        grid=(_N // _BN,),
        in_specs=[
            pl.BlockSpec(memory_space=pltpu.SMEM),
            pl.BlockSpec(memory_space=pltpu.SMEM),
            pl.BlockSpec((_T, _BN, _F), lambda i: (0, i, 0)),
            pl.BlockSpec((_T, _F, _F), lambda i: (0, 0, 0)),
        ],
        out_specs=pl.BlockSpec((_T, _BN, _F), lambda i: (0, i, 0)),
        out_shape=jax.ShapeDtypeStruct((_T, _N, _F), jnp.float32),
    )(minv, m, s1, w1)


def _layer2_body(minv_ref, s2_ref, w2_ref, u0_ref, u1_ref, p_ref, q_ref):
    a = [_round_bf(jnp.dot(s2_ref[t].astype(jnp.bfloat16),
                           w2_ref[t].astype(jnp.bfloat16),
                           preferred_element_type=jnp.float32))
         for t in range(_T)]
    for k in range(_T):
        z = _tmix_terms(minv_ref, a, k).astype(jnp.bfloat16)
        p_ref[k] = jnp.dot(z, u0_ref[...], preferred_element_type=jnp.float32)
        q_ref[k] = jnp.dot(z, u1_ref[...], preferred_element_type=jnp.float32)


def _layer2(minv, s2, w2, u0, u1):
    return pl.pallas_call(
        _layer2_body,
        grid=(_N // _BN,),
        in_specs=[
            pl.BlockSpec(memory_space=pltpu.SMEM),
            pl.BlockSpec((_T, _BN, _F), lambda i: (0, i, 0)),
            pl.BlockSpec((_T, _F, _F), lambda i: (0, 0, 0)),
            pl.BlockSpec((_F, _F), lambda i: (0, 0)),
            pl.BlockSpec((_F, _F), lambda i: (0, 0)),
        ],
        out_specs=[
            pl.BlockSpec((_T, _BN, _F), lambda i: (0, i, 0)),
            pl.BlockSpec((_T, _BN, _F), lambda i: (0, i, 0)),
        ],
        out_shape=[
            jax.ShapeDtypeStruct((_T, _N, _F), jnp.float32),
            jax.ShapeDtypeStruct((_T, _N, _F), jnp.float32),
        ],
    )(minv, s2, w2, u0, u1)


def _sc_mesh():
    return plsc.VectorSubcoreMesh(core_axis_name="c", subcore_axis_name="s",
                                  num_cores=_NC, num_subcores=_NS)


def _bucketize_body(at_t, at_s, at_d, at_v, src_o, val_o, dst_o, cnt_o,
                    tbuf, sbuf, dbuf, vbuf,
                    srcb0, srcb1, srcb2, srcb3, dstb0, dstb1, dstb2, dstb3,
                    valb0, valb1, valb2, valb3, cvec):
    c = lax.axis_index("c")
    s = lax.axis_index("s")
    widx = c * _NS + s
    srcbs = (srcb0, srcb1, srcb2, srcb3)
    dstbs = (dstb0, dstb1, dstb2, dstb3)
    valbs = (valb0, valb1, valb2, valb3)

    zero_i = jnp.zeros((16,), jnp.int32)
    zero_f = jnp.zeros((16,), jnp.float32)
    dump = jnp.full((16,), _DUMP, jnp.int32)

    def prefill(i, _):
        for p in range(_NP):
            srcbs[p][pl.ds(i * 16, 16)] = zero_i
            dstbs[p][pl.ds(i * 16, 16)] = dump
            valbs[p][pl.ds(i * 16, 16)] = zero_f
        return 0

    lax.fori_loop(0, (_CAP + 16) // 16, prefill, 0)

    def scan_chunk(ic, cnts):
        base_e = s * _STRIP + ic * _SCH
        pltpu.sync_copy(at_t.at[pl.ds(base_e, _SCH)], tbuf)
        pltpu.sync_copy(at_s.at[pl.ds(base_e, _SCH)], sbuf)
        pltpu.sync_copy(at_d.at[pl.ds(base_e, _SCH)], dbuf)
        pltpu.sync_copy(at_v.at[pl.ds(base_e, _SCH)], vbuf)

        lanes = lax.iota(jnp.int32, 16)

        def vec(i, cnts):
            t = tbuf[pl.ds(i * 16, 16)]
            sg = t * _N + sbuf[pl.ds(i * 16, 16)]
            dg = t * _N + dbuf[pl.ds(i * 16, 16)]
            v = vbuf[pl.ds(i * 16, 16)]
            out = []
            for p in range(_NP):
                off = dg - (p * _PASS + c * _WIN)
                m = (off >= 0) & (off < _WIN)
                cnt = cnts[p]
                mi = m.astype(jnp.int32)
                inc = plsc.cumsum(mi)
                pos = jnp.where(m, cnt + inc - 1, _CAP + lanes)
                plsc.store_scatter(srcbs[p], [pos], sg)
                plsc.store_scatter(dstbs[p], [pos], off)
                plsc.store_scatter(valbs[p], [pos], v)
                out.append(cnt + jnp.sum(mi))
            return tuple(out)

        return lax.fori_loop(0, _SCH // 16, vec, cnts)

    z = jnp.zeros((), jnp.int32)
    cnts = lax.fori_loop(0, _STRIP // _SCH, scan_chunk, (z,) * _NP)

    lanes = lax.iota(jnp.int32, 16)
    cv = jnp.zeros((16,), jnp.int32)
    for p in range(_NP):
        b = widx * _NP + p
        pltpu.sync_copy(srcbs[p].at[pl.ds(0, _CAP)], src_o.at[b])
        pltpu.sync_copy(valbs[p].at[pl.ds(0, _CAP)], val_o.at[b])
        for ch in range(_NCHK):
            pltpu.sync_copy(dstbs[p].at[pl.ds(ch * 128, 128)], dst_o.at[b, ch])
        nch = (cnts[p] + 127) // 128
        cv = jnp.where(lanes == p, jnp.full((16,), nch, jnp.int32), cv)
    cvec[...] = cv
    pltpu.sync_copy(cvec, cnt_o.at[widx])


def _bucketize(at_t, at_s, at_d, at_v):
    f = pl.kernel(
        _bucketize_body,
        out_type=[
            jax.ShapeDtypeStruct((_NW * _NP, _CAP), jnp.int32),
            jax.ShapeDtypeStruct((_NW * _NP, _CAP), jnp.float32),
            jax.ShapeDtypeStruct((_NW * _NP, _NCHK, 128), jnp.int32),
            jax.ShapeDtypeStruct((_NW, 16), jnp.int32),
        ],
        mesh=_sc_mesh(),
        compiler_params=pltpu.CompilerParams(needs_layout_passes=False),
        scratch_types=[
            pltpu.VMEM((_SCH,), jnp.int32),
            pltpu.VMEM((_SCH,), jnp.int32),
            pltpu.VMEM((_SCH,), jnp.int32),
            pltpu.VMEM((_SCH,), jnp.float32),
        ] + [pltpu.VMEM((_CAP + 16,), jnp.int32)] * (2 * _NP)
          + [pltpu.VMEM((_CAP + 16,), jnp.float32)] * _NP
          + [pltpu.VMEM((16,), jnp.int32)],
    )
    return f(at_t, at_s, at_d, at_v)


def _spmm_body(xt, zeros_h, src_t, val_t, dst_t, cnt_t, out,
               srcv, valv, dstv, cntv, rows0, rows1, acc, sem0, sem1):
    c = lax.axis_index("c")
    s = lax.axis_index("s")
    widx = c * _NS + s

    pltpu.sync_copy(cnt_t.at[widx], cntv)
    cv = cntv[...]
    lanes = lax.iota(jnp.int32, 16)

    def start_gather(j, buf, sem):
        pltpu.async_copy(xt.at[srcv.at[pl.ds(j * 128, 128)]], buf, sem)

    def wait_gather(buf, sem):
        pltpu.make_async_copy(xt.at[pl.ds(0, 128)], buf, sem).wait()

    def scale_scatter(buf, j):
        def scale(r, _):
            vb = plsc.load_gather(
                valv, [jnp.full((16,), j * 128 + r, jnp.int32)])
            for f in range(8):
                buf[r, pl.ds(f * 16, 16)] = buf[r, pl.ds(f * 16, 16)] * vb
            return 0

        lax.fori_loop(0, 128, scale, 0)
        pltpu.sync_copy(buf, acc.at[dstv.at[j]], add=True)

    for p in range(_NP):
        b = widx * _NP + p
        pltpu.sync_copy(src_t.at[b], srcv)
        pltpu.sync_copy(val_t.at[b], valv)
        pltpu.sync_copy(dst_t.at[b], dstv)
        pltpu.sync_copy(zeros_h, acc.at[pl.ds(s * (_ACC // _NS), _ACC // _NS)])
        plsc.subcore_barrier()

        nch = jnp.sum(jnp.where(lanes == p, cv, 0))

        @pl.when(nch > 0)
        def _():
            start_gather(0, rows0, sem0)

        def pair(jp, _):
            j0 = 2 * jp
            j1 = j0 + 1

            @pl.when(j1 < nch)
            def _():
                start_gather(j1, rows1, sem1)

            wait_gather(rows0, sem0)
            scale_scatter(rows0, j0)

            @pl.when(j1 < nch)
            def _():
                @pl.when(j1 + 1 < nch)
                def _():
                    start_gather(j1 + 1, rows0, sem0)

                wait_gather(rows1, sem1)
                scale_scatter(rows1, j1)

            return 0

        lax.fori_loop(0, (nch + 1) // 2, pair, 0)
        plsc.subcore_barrier()

        tw = _WIN // _NS
        gbase = p * _PASS + c * _WIN + s * tw
        for i in range(tw // 128):
            pltpu.sync_copy(acc.at[pl.ds(s * tw + i * 128, 128)],
                            out.at[pl.ds(gbase + i * 128, 128)])
        plsc.subcore_barrier()


def _spmm(xt, zeros_h, src_t, val_t, dst_t, cnt_t):
    f = pl.kernel(
        _spmm_body,
        out_type=jax.ShapeDtypeStruct((_RPAD, _F), jnp.float32),
        mesh=_sc_mesh(),
        compiler_params=pltpu.CompilerParams(needs_layout_passes=False),
        scratch_types=[
            pltpu.VMEM((_CAP,), jnp.int32),
            pltpu.VMEM((_CAP,), jnp.float32),
            pltpu.VMEM((_NCHK, 128), jnp.int32),
            pltpu.VMEM((16,), jnp.int32),
            pltpu.VMEM((128, _F), jnp.float32),
            pltpu.VMEM((128, _F), jnp.float32),
            pltpu.VMEM_SHARED((_ACC, _F), jnp.float32),
            pltpu.SemaphoreType.DMA,
            pltpu.SemaphoreType.DMA,
        ],
    )
    return f(xt, zeros_h, src_t, val_t, dst_t, cnt_t)


_EW = _E // _NW
_ECH = 80


def _edge_body(p_t, q_t, et, es, eg, out, tb, sb, gb, esn, etn, rows, sem):
    c = lax.axis_index("c")
    s = lax.axis_index("s")
    widx = c * _NS + s
    base = widx * _EW

    pltpu.sync_copy(et.at[pl.ds(base, _EW)], tb)
    pltpu.sync_copy(es.at[pl.ds(base, _EW)], sb)
    pltpu.sync_copy(eg.at[pl.ds(base, _EW)], gb)

    def cvt(i, _):
        t = tb[pl.ds(i * 16, 16)]
        esn[pl.ds(i * 16, 16)] = t * _N + sb[pl.ds(i * 16, 16)]
        etn[pl.ds(i * 16, 16)] = t * _N + gb[pl.ds(i * 16, 16)]
        return 0

    lax.fori_loop(0, _EW // 16, cvt, 0)

    def chunk(j, _):
        pltpu.async_copy(p_t.at[esn.at[pl.ds(j * _ECH, _ECH)]], rows,
                         sem).wait()
        pltpu.async_copy(q_t.at[etn.at[pl.ds(j * _ECH, _ECH)]], rows,
                         sem, add=True).wait()
        pltpu.sync_copy(rows, out.at[pl.ds(base + j * _ECH, _ECH)])
        return 0

    lax.fori_loop(0, _EW // _ECH, chunk, 0)


def _edge_out(p_t, q_t, et, es, eg):
    f = pl.kernel(
        _edge_body,
        out_type=jax.ShapeDtypeStruct((_E, _F), jnp.float32),
        mesh=_sc_mesh(),
        compiler_params=pltpu.CompilerParams(needs_layout_passes=False),
        scratch_types=[
            pltpu.VMEM((_EW,), jnp.int32),
            pltpu.VMEM((_EW,), jnp.int32),
            pltpu.VMEM((_EW,), jnp.int32),
            pltpu.VMEM((_EW,), jnp.int32),
            pltpu.VMEM((_EW,), jnp.int32),
            pltpu.VMEM((_ECH, _F), jnp.float32),
            pltpu.SemaphoreType.DMA,
        ],
    )
    return f(p_t, q_t, et, es, eg)


def kernel(X, at_time, at_src, at_dst, at_val, edge_time, edge_src, edge_trg,
           M, Minv, W1, W2, U):
    mr = lax.optimization_barrier(M.astype(jnp.bfloat16)).astype(jnp.float32)
    minvr = lax.optimization_barrier(
        Minv.astype(jnp.bfloat16)).astype(jnp.float32)
    xt = _mix(mr, X)
    zeros_h = jnp.zeros((_ACC // _NS, _F), jnp.float32)
    src_t, val_t, dst_t, cnt_t = _bucketize(at_time, at_src, at_dst, at_val)
    s1 = _spmm(xt.reshape(_R, _F), zeros_h, src_t, val_t, dst_t, cnt_t)
    yt = _layer1(minvr, mr, s1[:_R].reshape(_T, _N, _F), W1)
    s2 = _spmm(yt.reshape(_R, _F), zeros_h, src_t, val_t, dst_t, cnt_t)
    p, q = _layer2(minvr, s2[:_R].reshape(_T, _N, _F), W2,
                   U[:_F].astype(jnp.bfloat16), U[_F:].astype(jnp.bfloat16))
    return _edge_out(p.reshape(_R, _F), q.reshape(_R, _F),
                     edge_time, edge_src, edge_trg)


# speedup vs baseline: 3.4219x; 1.0748x over previous
import functools

import jax
import jax.numpy as jnp
from jax import lax
from jax.experimental import pallas as pl
from jax.experimental.pallas import tpu as pltpu
from jax.experimental.pallas import tpu_sc as plsc

_T = 8
_N = 10000
_E = 320000
_F = 128
_R = _T * _N

_NC = 2
_NS = 16
_NW = _NC * _NS

_WIN = 10240
_PASS = _WIN * _NC
_NP = 4
_RPAD = _PASS * _NP
_ACC = 10368
_DUMP = _WIN
_CAP = 3584
_NCHK = _CAP // 128
_STRIP = _E // _NS
_SCH = 2000

_BN = 1000


def _round_bf(x):
    return x.astype(jnp.bfloat16).astype(jnp.float32)


def _mix_body(m_ref, x_ref, o_ref):
    xs = [_round_bf(x_ref[k]) for k in range(_T)]
    for t in range(_T):
        acc = xs[0] * m_ref[t, 0]
        for k in range(1, _T):
            acc = acc + xs[k] * m_ref[t, k]
        o_ref[t] = acc


def _mix(m, x):
    return pl.pallas_call(
        _mix_body,
        grid=(_N // _BN,),
        in_specs=[
            pl.BlockSpec(memory_space=pltpu.SMEM),
            pl.BlockSpec((_T, _BN, _F), lambda i: (0, i, 0)),
        ],
        out_specs=pl.BlockSpec((_T, _BN, _F), lambda i: (0, i, 0)),
        out_shape=jax.ShapeDtypeStruct((_T, _N, _F), jnp.float32),
    )(m, x)


def _tmix_terms(coef_ref, arrs, k):
    acc = arrs[0] * coef_ref[k, 0]
    for t in range(1, _T):
        acc = acc + arrs[t] * coef_ref[k, t]
    return acc


def _layer1_body(minv_ref, m_ref, s1_ref, w1_ref, o_ref):
    a = [_round_bf(jnp.dot(s1_ref[t].astype(jnp.bfloat16),
                           w1_ref[t].astype(jnp.bfloat16),
                           preferred_element_type=jnp.float32))
         for t in range(_T)]
    y = [_round_bf(jnp.maximum(_tmix_terms(minv_ref, a, k), 0.0))
         for k in range(_T)]
    for t in range(_T):
        o_ref[t] = _tmix_terms(m_ref, y, t)


def _layer1(minv, m, s1, w1):
    return pl.pallas_call(
        _layer1_body,
        grid=(_N // _BN,),
        in_specs=[
            pl.BlockSpec(memory_space=pltpu.SMEM),
            pl.BlockSpec(memory_space=pltpu.SMEM),
            pl.BlockSpec((_T, _BN, _F), lambda i: (0, i, 0)),
            pl.BlockSpec((_T, _F, _F), lambda i: (0, 0, 0)),
        ],
        out_specs=pl.BlockSpec((_T, _BN, _F), lambda i: (0, i, 0)),
        out_shape=jax.ShapeDtypeStruct((_T, _N, _F), jnp.float32),
    )(minv, m, s1, w1)


def _layer2_body(minv_ref, s2_ref, w2_ref, u0_ref, u1_ref, p_ref, q_ref):
    a = [_round_bf(jnp.dot(s2_ref[t].astype(jnp.bfloat16),
                           w2_ref[t].astype(jnp.bfloat16),
                           preferred_element_type=jnp.float32))
         for t in range(_T)]
    for k in range(_T):
        z = _tmix_terms(minv_ref, a, k).astype(jnp.bfloat16)
        p_ref[k] = jnp.dot(z, u0_ref[...], preferred_element_type=jnp.float32)
        q_ref[k] = jnp.dot(z, u1_ref[...], preferred_element_type=jnp.float32)


def _layer2(minv, s2, w2, u0, u1):
    return pl.pallas_call(
        _layer2_body,
        grid=(_N // _BN,),
        in_specs=[
            pl.BlockSpec(memory_space=pltpu.SMEM),
            pl.BlockSpec((_T, _BN, _F), lambda i: (0, i, 0)),
            pl.BlockSpec((_T, _F, _F), lambda i: (0, 0, 0)),
            pl.BlockSpec((_F, _F), lambda i: (0, 0)),
            pl.BlockSpec((_F, _F), lambda i: (0, 0)),
        ],
        out_specs=[
            pl.BlockSpec((_T, _BN, _F), lambda i: (0, i, 0)),
            pl.BlockSpec((_T, _BN, _F), lambda i: (0, i, 0)),
        ],
        out_shape=[
            jax.ShapeDtypeStruct((_T, _N, _F), jnp.float32),
            jax.ShapeDtypeStruct((_T, _N, _F), jnp.float32),
        ],
    )(minv, s2, w2, u0, u1)


def _sc_mesh():
    return plsc.VectorSubcoreMesh(core_axis_name="c", subcore_axis_name="s",
                                  num_cores=_NC, num_subcores=_NS)


def _bucketize_body(at_t, at_s, at_d, at_v, src_o, val_o, dst_o, cnt_o,
                    tbuf, sbuf, dbuf, vbuf,
                    srcb0, srcb1, srcb2, srcb3, dstb0, dstb1, dstb2, dstb3,
                    valb0, valb1, valb2, valb3, cvec):
    c = lax.axis_index("c")
    s = lax.axis_index("s")
    widx = c * _NS + s
    srcbs = (srcb0, srcb1, srcb2, srcb3)
    dstbs = (dstb0, dstb1, dstb2, dstb3)
    valbs = (valb0, valb1, valb2, valb3)

    zero_i = jnp.zeros((16,), jnp.int32)
    zero_f = jnp.zeros((16,), jnp.float32)
    dump = jnp.full((16,), _DUMP, jnp.int32)

    def prefill(i, _):
        for p in range(_NP):
            srcbs[p][pl.ds(i * 16, 16)] = zero_i
            dstbs[p][pl.ds(i * 16, 16)] = dump
            valbs[p][pl.ds(i * 16, 16)] = zero_f
        return 0

    lax.fori_loop(0, (_CAP + 16) // 16, prefill, 0)

    def scan_chunk(ic, cnts):
        base_e = s * _STRIP + ic * _SCH
        pltpu.sync_copy(at_t.at[pl.ds(base_e, _SCH)], tbuf)
        pltpu.sync_copy(at_s.at[pl.ds(base_e, _SCH)], sbuf)
        pltpu.sync_copy(at_d.at[pl.ds(base_e, _SCH)], dbuf)
        pltpu.sync_copy(at_v.at[pl.ds(base_e, _SCH)], vbuf)

        lanes = lax.iota(jnp.int32, 16)

        def vec(i, cnts):
            t = tbuf[pl.ds(i * 16, 16)]
            sg = t * _N + sbuf[pl.ds(i * 16, 16)]
            dg = t * _N + dbuf[pl.ds(i * 16, 16)]
            v = vbuf[pl.ds(i * 16, 16)]
            out = []
            for p in range(_NP):
                off = dg - (p * _PASS + c * _WIN)
                m = (off >= 0) & (off < _WIN)
                cnt = cnts[p]
                mi = m.astype(jnp.int32)
                inc = plsc.cumsum(mi)
                pos = jnp.where(m, cnt + inc - 1, _CAP + lanes)
                plsc.store_scatter(srcbs[p], [pos], sg)
                plsc.store_scatter(dstbs[p], [pos], off)
                plsc.store_scatter(valbs[p], [pos], v)
                out.append(cnt + jnp.sum(mi))
            return tuple(out)

        return lax.fori_loop(0, _SCH // 16, vec, cnts)

    z = jnp.zeros((), jnp.int32)
    cnts = lax.fori_loop(0, _STRIP // _SCH, scan_chunk, (z,) * _NP)

    lanes = lax.iota(jnp.int32, 16)
    cv = jnp.zeros((16,), jnp.int32)
    for p in range(_NP):
        b = widx * _NP + p
        pltpu.sync_copy(srcbs[p].at[pl.ds(0, _CAP)], src_o.at[b])
        pltpu.sync_copy(valbs[p].at[pl.ds(0, _CAP)], val_o.at[b])
        for ch in range(_NCHK):
            pltpu.sync_copy(dstbs[p].at[pl.ds(ch * 128, 128)], dst_o.at[b, ch])
        nch = (cnts[p] + 127) // 128
        cv = jnp.where(lanes == p, jnp.full((16,), nch, jnp.int32), cv)
    cvec[...] = cv
    pltpu.sync_copy(cvec, cnt_o.at[widx])


def _bucketize(at_t, at_s, at_d, at_v):
    f = pl.kernel(
        _bucketize_body,
        out_type=[
            jax.ShapeDtypeStruct((_NW * _NP, _CAP), jnp.int32),
            jax.ShapeDtypeStruct((_NW * _NP, _CAP), jnp.float32),
            jax.ShapeDtypeStruct((_NW * _NP, _NCHK, 128), jnp.int32),
            jax.ShapeDtypeStruct((_NW, 16), jnp.int32),
        ],
        mesh=_sc_mesh(),
        compiler_params=pltpu.CompilerParams(needs_layout_passes=False),
        scratch_types=[
            pltpu.VMEM((_SCH,), jnp.int32),
            pltpu.VMEM((_SCH,), jnp.int32),
            pltpu.VMEM((_SCH,), jnp.int32),
            pltpu.VMEM((_SCH,), jnp.float32),
        ] + [pltpu.VMEM((_CAP + 16,), jnp.int32)] * (2 * _NP)
          + [pltpu.VMEM((_CAP + 16,), jnp.float32)] * _NP
          + [pltpu.VMEM((16,), jnp.int32)],
    )
    return f(at_t, at_s, at_d, at_v)


def _spmm_body(xt, zeros_h, src_t, val_t, dst_t, cnt_t, out,
               srcv, valv, dstv, cntv, rows0, rows1, acc, sem0, sem1):
    c = lax.axis_index("c")
    s = lax.axis_index("s")
    widx = c * _NS + s

    pltpu.sync_copy(cnt_t.at[widx], cntv)
    cv = cntv[...]
    lanes = lax.iota(jnp.int32, 16)

    def start_gather(j, buf, sem):
        pltpu.async_copy(xt.at[srcv.at[pl.ds(j * 128, 128)]], buf, sem)

    def wait_gather(buf, sem):
        pltpu.make_async_copy(xt.at[pl.ds(0, 128)], buf, sem).wait()

    def scale_scatter(buf, j):
        def scale(r, _):
            vb = plsc.load_gather(
                valv, [jnp.full((16,), j * 128 + r, jnp.int32)])
            for f in range(8):
                buf[r, pl.ds(f * 16, 16)] = buf[r, pl.ds(f * 16, 16)] * vb
            return 0

        lax.fori_loop(0, 128, scale, 0)
        pltpu.sync_copy(buf, acc.at[dstv.at[j]], add=True)

    for p in range(_NP):
        b = widx * _NP + p
        pltpu.sync_copy(src_t.at[b], srcv)
        pltpu.sync_copy(val_t.at[b], valv)
        pltpu.sync_copy(dst_t.at[b], dstv)
        pltpu.sync_copy(zeros_h, acc.at[pl.ds(s * (_ACC // _NS), _ACC // _NS)])
        plsc.subcore_barrier()

        nch = jnp.sum(jnp.where(lanes == p, cv, 0))

        @pl.when(nch > 0)
        def _():
            start_gather(0, rows0, sem0)

        def pair(jp, _):
            j0 = 2 * jp
            j1 = j0 + 1

            @pl.when(j1 < nch)
            def _():
                start_gather(j1, rows1, sem1)

            wait_gather(rows0, sem0)
            scale_scatter(rows0, j0)

            @pl.when(j1 < nch)
            def _():
                @pl.when(j1 + 1 < nch)
                def _():
                    start_gather(j1 + 1, rows0, sem0)

                wait_gather(rows1, sem1)
                scale_scatter(rows1, j1)

            return 0

        lax.fori_loop(0, (nch + 1) // 2, pair, 0)
        plsc.subcore_barrier()

        tw = _WIN // _NS
        gbase = p * _PASS + c * _WIN + s * tw
        for i in range(tw // 128):
            pltpu.sync_copy(acc.at[pl.ds(s * tw + i * 128, 128)],
                            out.at[pl.ds(gbase + i * 128, 128)])
        plsc.subcore_barrier()


def _spmm(xt, zeros_h, src_t, val_t, dst_t, cnt_t):
    f = pl.kernel(
        _spmm_body,
        out_type=jax.ShapeDtypeStruct((_RPAD, _F), jnp.float32),
        mesh=_sc_mesh(),
        compiler_params=pltpu.CompilerParams(needs_layout_passes=False),
        scratch_types=[
            pltpu.VMEM((_CAP,), jnp.int32),
            pltpu.VMEM((_CAP,), jnp.float32),
            pltpu.VMEM((_NCHK, 128), jnp.int32),
            pltpu.VMEM((16,), jnp.int32),
            pltpu.VMEM((128, _F), jnp.float32),
            pltpu.VMEM((128, _F), jnp.float32),
            pltpu.VMEM_SHARED((_ACC, _F), jnp.float32),
            pltpu.SemaphoreType.DMA,
            pltpu.SemaphoreType.DMA,
        ],
    )
    return f(xt, zeros_h, src_t, val_t, dst_t, cnt_t)


_EW = _E // _NW
_ECH = 80


def _edge_body(p_t, q_t, et, es, eg, out, tb, sb, gb, esn, etn,
               rows0, rows1, sem0, sem1):
    c = lax.axis_index("c")
    s = lax.axis_index("s")
    widx = c * _NS + s
    base = widx * _EW
    nch = _EW // _ECH

    pltpu.sync_copy(et.at[pl.ds(base, _EW)], tb)
    pltpu.sync_copy(es.at[pl.ds(base, _EW)], sb)
    pltpu.sync_copy(eg.at[pl.ds(base, _EW)], gb)

    def cvt(i, _):
        t = tb[pl.ds(i * 16, 16)]
        esn[pl.ds(i * 16, 16)] = t * _N + sb[pl.ds(i * 16, 16)]
        etn[pl.ds(i * 16, 16)] = t * _N + gb[pl.ds(i * 16, 16)]
        return 0

    lax.fori_loop(0, _EW // 16, cvt, 0)

    def start_p(j, buf, sem):
        pltpu.async_copy(p_t.at[esn.at[pl.ds(j * _ECH, _ECH)]], buf, sem)

    def waitb(buf, sem):
        pltpu.make_async_copy(p_t.at[pl.ds(0, _ECH)], buf, sem).wait()

    def q_add_store(j, buf, sem):
        pltpu.async_copy(q_t.at[etn.at[pl.ds(j * _ECH, _ECH)]], buf, sem,
                         add=True)
        waitb(buf, sem)
        pltpu.sync_copy(buf, out.at[pl.ds(base + j * _ECH, _ECH)])

    start_p(0, rows0, sem0)

    def pair(jp, _):
        j0 = 2 * jp
        j1 = j0 + 1
        start_p(j1, rows1, sem1)
        waitb(rows0, sem0)
        q_add_store(j0, rows0, sem0)

        @pl.when(j1 + 1 < nch)
        def _():
            start_p(j1 + 1, rows0, sem0)

        waitb(rows1, sem1)
        q_add_store(j1, rows1, sem1)
        return 0

    lax.fori_loop(0, nch // 2, pair, 0)
    waitb(rows0, sem0)
    q_add_store(nch - 1, rows0, sem0)


def _edge_out(p_t, q_t, et, es, eg):
    f = pl.kernel(
        _edge_body,
        out_type=jax.ShapeDtypeStruct((_E, _F), jnp.float32),
        mesh=_sc_mesh(),
        compiler_params=pltpu.CompilerParams(needs_layout_passes=False),
        scratch_types=[
            pltpu.VMEM((_EW,), jnp.int32),
            pltpu.VMEM((_EW,), jnp.int32),
            pltpu.VMEM((_EW,), jnp.int32),
            pltpu.VMEM((_EW,), jnp.int32),
            pltpu.VMEM((_EW,), jnp.int32),
            pltpu.VMEM((_ECH, _F), jnp.float32),
            pltpu.VMEM((_ECH, _F), jnp.float32),
            pltpu.SemaphoreType.DMA,
            pltpu.SemaphoreType.DMA,
        ],
    )
    return f(p_t, q_t, et, es, eg)


def kernel(X, at_time, at_src, at_dst, at_val, edge_time, edge_src, edge_trg,
           M, Minv, W1, W2, U):
    mr = lax.optimization_barrier(M.astype(jnp.bfloat16)).astype(jnp.float32)
    minvr = lax.optimization_barrier(
        Minv.astype(jnp.bfloat16)).astype(jnp.float32)
    xt = _mix(mr, X)
    zeros_h = jnp.zeros((_ACC // _NS, _F), jnp.float32)
    src_t, val_t, dst_t, cnt_t = _bucketize(at_time, at_src, at_dst, at_val)
    s1 = _spmm(xt.reshape(_R, _F), zeros_h, src_t, val_t, dst_t, cnt_t)
    yt = _layer1(minvr, mr, s1[:_R].reshape(_T, _N, _F), W1)
    s2 = _spmm(yt.reshape(_R, _F), zeros_h, src_t, val_t, dst_t, cnt_t)
    p, q = _layer2(minvr, s2[:_R].reshape(_T, _N, _F), W2,
                   U[:_F].astype(jnp.bfloat16), U[_F:].astype(jnp.bfloat16))
    return _edge_out(p.reshape(_R, _F), q.reshape(_R, _F),
                     edge_time, edge_src, edge_trg)
